# baseline (speedup 1.0000x reference)
import functools

import jax
import jax.numpy as jnp
from jax import lax
from jax.experimental import pallas as pl
from jax.experimental.pallas import tpu as pltpu
from jax.experimental.pallas import tpu_sc as plsc

L = 16
SUB = 8
LANES = 128


@functools.lru_cache(maxsize=None)
def _make_gather(batch, nf, V, D):
    info = plsc.get_sparse_core_info()
    NC, NS = info.num_cores, info.num_subcores
    NW = NC * NS
    EH = D // SUB
    SH = batch // LANES
    assert SH % NW == 0
    spw = SH // NW
    r_per_w = (batch // NW) * nf
    blk = LANES * nf
    gch = blk // 8
    ntiles = nf * EH

    mesh = plsc.VectorSubcoreMesh(core_axis_name="c", subcore_axis_name="s")

    @functools.partial(
        pl.kernel,
        mesh=mesh,
        out_type=jax.ShapeDtypeStruct((nf, EH, SH, SUB, LANES), jnp.float32),
        scratch_types=[
            pltpu.VMEM((r_per_w,), jnp.int32),
            pltpu.VMEM((blk, D), jnp.float32),
            pltpu.VMEM((2, SUB, LANES), jnp.float32),
            pltpu.SemaphoreType.DMA,
            pltpu.SemaphoreType.DMA,
            pltpu.SemaphoreType.DMA,
        ],
        compiler_params=pltpu.CompilerParams(
            use_tc_tiling_on_sc=False, needs_layout_passes=False
        ),
    )
    def gather_kernel(idx_hbm, table_hbm, out_hbm, idx_v, rows, tiles,
                      sem_g, sem_o0, sem_o1):
        sem_o = [sem_o0, sem_o1]
        wid = lax.axis_index("s") * NC + lax.axis_index("c")
        pltpu.sync_copy(idx_hbm.at[pl.ds(wid * r_per_w, r_per_w)], idx_v)

        lane = lax.iota(jnp.int32, L)
        rbases = [lane * nf + sl0 * L * nf for sl0 in range(SUB)]
        zeros = lane * 0

        def drain(slot):
            pltpu.make_async_copy(
                tiles.at[slot], out_hbm.at[0].at[0].at[0], sem_o[slot]
            ).wait()

        for shl in range(spw):
            sh = wid * spw + shl
            handles = [
                pltpu.async_copy(
                    table_hbm.at[
                        idx_v.at[pl.ds(shl * blk + c * gch, gch)]
                    ],
                    rows.at[pl.ds(c * gch, gch)],
                    sem_g,
                )
                for c in range(8)
            ]
            for h in handles:
                h.wait()

            def tbody(tt, carry):
                for slot in range(2):
                    t = tt * 2 + slot
                    f = t // EH
                    eh = t % EH

                    @pl.when(tt >= 1)
                    def _():
                        drain(slot)

                    rowidx = [rb + f for rb in rbases]
                    cols = [zeros + (eh * SUB + el) for el in range(SUB)]
                    for el2 in range(SUB // 4):
                        els = tuple(range(4 * el2, 4 * el2 + 4))
                        vecs = [
                            plsc.load_gather(rows, [rowidx[sl0], cols[el]])
                            for el in els
                            for sl0 in range(SUB)
                        ]
                        k = 0
                        for el in els:
                            for sl0 in range(SUB):
                                tiles.at[slot].at[el][
                                    pl.ds(sl0 * L, L)
                                ] = vecs[k]
                                k += 1
                    pltpu.async_copy(
                        tiles.at[slot],
                        out_hbm.at[f].at[eh].at[sh],
                        sem_o[slot],
                    )
                return carry

            lax.fori_loop(0, ntiles // 2, tbody, 0)
            for slot in range(2):
                drain(slot)

    return gather_kernel


def kernel(x, weight):
    batch, nf = x.shape
    V, D = weight.shape
    idx = x.reshape(batch * nf)
    out5d = _make_gather(batch, nf, V, D)(idx, weight)
    t = jnp.transpose(out5d, (2, 4, 0, 1, 3))
    return t.reshape(batch, nf, D)


# speedup vs baseline: 1.0350x; 1.0350x over previous
import functools

import jax
import jax.numpy as jnp
from jax import lax
from jax.experimental import pallas as pl
from jax.experimental.pallas import tpu as pltpu
from jax.experimental.pallas import tpu_sc as plsc

L = 16
SUB = 8
LANES = 128


@functools.lru_cache(maxsize=None)
def _make_gather(batch, nf, V, D):
    info = plsc.get_sparse_core_info()
    NC, NS = info.num_cores, info.num_subcores
    NW = NC * NS
    EH = D // SUB
    SH = batch // LANES
    assert SH % NW == 0
    spw = SH // NW
    r_per_w = (batch // NW) * nf
    blk = LANES * nf
    nr = spw * nf

    mesh = plsc.VectorSubcoreMesh(core_axis_name="c", subcore_axis_name="s")

    @functools.partial(
        pl.kernel,
        mesh=mesh,
        out_type=jax.ShapeDtypeStruct((nf, EH, SH, SUB, LANES), jnp.float32),
        scratch_types=[
            pltpu.VMEM((r_per_w,), jnp.int32),
            pltpu.VMEM((nr, LANES), jnp.int32),
            pltpu.VMEM((2, LANES, D), jnp.float32),
            pltpu.VMEM((2, EH, SUB, LANES), jnp.float32),
            pltpu.SemaphoreType.DMA,
            pltpu.SemaphoreType.DMA,
            pltpu.SemaphoreType.DMA,
            pltpu.SemaphoreType.DMA,
        ],
        compiler_params=pltpu.CompilerParams(
            use_tc_tiling_on_sc=False, needs_layout_passes=False
        ),
    )
    def gather_kernel(idx_hbm, table_hbm, out_hbm, idx_v, idx_fm, rows,
                      tiles, sem_g0, sem_g1, sem_o0, sem_o1):
        sem_g = [sem_g0, sem_g1]
        sem_o = [sem_o0, sem_o1]
        wid = lax.axis_index("s") * NC + lax.axis_index("c")
        pltpu.sync_copy(idx_hbm.at[pl.ds(wid * r_per_w, r_per_w)], idx_v)

        lane = lax.iota(jnp.int32, L)
        lane_nf = lane * nf
        zeros = lane * 0
        rowidx = [lane + sl0 * L for sl0 in range(SUB)]

        for shl in range(spw):
            def rbody(f, carry, shl=shl):
                base = shl * blk + f
                vecs = [
                    plsc.load_gather(
                        idx_v, [lane_nf + (base + sl0 * L * nf)]
                    )
                    for sl0 in range(SUB)
                ]
                for sl0 in range(SUB):
                    idx_fm.at[shl * nf + f][pl.ds(sl0 * L, L)] = vecs[sl0]
                return carry

            lax.fori_loop(0, nf, rbody, 0)

        def fire(r, slot):
            pltpu.async_copy(
                table_hbm.at[idx_fm.at[r]], rows.at[slot], sem_g[slot]
            )

        fire(0, 0)
        fire(1, 1)

        def body(rr, carry):
            for slot in range(2):
                r = rr * 2 + slot
                shl = r // nf
                f = r % nf
                sh = wid * spw + shl

                @pl.when(rr >= 1)
                def _():
                    for _ in range(EH):
                        pltpu.make_async_copy(
                            tiles.at[slot].at[0],
                            out_hbm.at[0].at[0].at[0],
                            sem_o[slot],
                        ).wait()

                pltpu.make_async_copy(
                    table_hbm.at[pl.ds(0, LANES)], rows.at[slot],
                    sem_g[slot],
                ).wait()

                for eh in range(EH):
                    for el2 in range(SUB // 2):
                        els = (2 * el2, 2 * el2 + 1)
                        cols = [zeros + (eh * SUB + el) for el in els]
                        vecs = [
                            plsc.load_gather(
                                rows.at[slot], [rowidx[sl0], cols[i]]
                            )
                            for i in range(2)
                            for sl0 in range(SUB)
                        ]
                        k = 0
                        for el in els:
                            for sl0 in range(SUB):
                                tiles.at[slot].at[eh].at[el][
                                    pl.ds(sl0 * L, L)
                                ] = vecs[k]
                                k += 1
                    pltpu.async_copy(
                        tiles.at[slot].at[eh],
                        out_hbm.at[f].at[eh].at[sh],
                        sem_o[slot],
                    )

                @pl.when(r + 2 < nr)
                def _():
                    fire(r + 2, slot)
            return carry

        lax.fori_loop(0, nr // 2, body, 0)

        for slot in range(2):
            for _ in range(EH):
                pltpu.make_async_copy(
                    tiles.at[slot].at[0], out_hbm.at[0].at[0].at[0],
                    sem_o[slot],
                ).wait()

    return gather_kernel


def kernel(x, weight):
    batch, nf = x.shape
    V, D = weight.shape
    idx = x.reshape(batch * nf)
    out5d = _make_gather(batch, nf, V, D)(idx, weight)
    t = jnp.transpose(out5d, (2, 4, 0, 1, 3))
    return t.reshape(batch, nf, D)


# speedup vs baseline: 1.1488x; 1.1099x over previous
import functools

import jax
import jax.numpy as jnp
from jax import lax
from jax.experimental import pallas as pl
from jax.experimental.pallas import tpu as pltpu
from jax.experimental.pallas import tpu_sc as plsc

L = 16
SUB = 8
LANES = 128


@functools.lru_cache(maxsize=None)
def _make_gather(batch, nf, V, D):
    info = plsc.get_sparse_core_info()
    NC, NS = info.num_cores, info.num_subcores
    NW = NC * NS
    EH = D // SUB
    SH = batch // LANES
    assert SH % NW == 0
    spw = SH // NW
    r_per_w = (batch // NW) * nf
    blk = LANES * nf
    nr = spw * nf

    mesh = plsc.VectorSubcoreMesh(core_axis_name="c", subcore_axis_name="s")

    @functools.partial(
        pl.kernel,
        mesh=mesh,
        out_type=jax.ShapeDtypeStruct((nf, EH, SH, SUB, LANES), jnp.float32),
        scratch_types=[
            pltpu.VMEM((r_per_w,), jnp.int32),
            pltpu.VMEM((nr, LANES), jnp.int32),
            pltpu.VMEM((2, LANES, D), jnp.float32),
            pltpu.VMEM((2, EH * SUB, LANES), jnp.float32),
            pltpu.SemaphoreType.DMA,
            pltpu.SemaphoreType.DMA,
            pltpu.SemaphoreType.DMA,
            pltpu.SemaphoreType.DMA,
        ],
        compiler_params=pltpu.CompilerParams(
            use_tc_tiling_on_sc=False, needs_layout_passes=False
        ),
    )
    def gather_kernel(idx_hbm, table_hbm, out_hbm, idx_v, idx_fm, rows,
                      tiles, sem_g0, sem_g1, sem_o0, sem_o1):
        sem_g = [sem_g0, sem_g1]
        sem_o = [sem_o0, sem_o1]
        wid = lax.axis_index("s") * NC + lax.axis_index("c")
        pltpu.sync_copy(idx_hbm.at[pl.ds(wid * r_per_w, r_per_w)], idx_v)

        lane = lax.iota(jnp.int32, L)
        lane_nf = lane * nf
        zeros = lane * 0
        rowidx = [lane + sl0 * L for sl0 in range(SUB)]

        for shl in range(spw):
            def rbody(f, carry, shl=shl):
                base = shl * blk + f
                vecs = [
                    plsc.load_gather(
                        idx_v, [lane_nf + (base + sl0 * L * nf)]
                    )
                    for sl0 in range(SUB)
                ]
                for sl0 in range(SUB):
                    idx_fm.at[shl * nf + f][pl.ds(sl0 * L, L)] = vecs[sl0]
                return carry

            lax.fori_loop(0, nf, rbody, 0)

        def fire(r, slot):
            pltpu.async_copy(
                table_hbm.at[idx_fm.at[r]], rows.at[slot], sem_g[slot]
            )

        fire(0, 0)
        fire(1, 1)

        def body(rr, carry):
            for slot in range(2):
                r = rr * 2 + slot
                shl = r // nf
                f = r % nf
                sh = wid * spw + shl

                @pl.when(rr >= 1)
                def _():
                    for _ in range(EH):
                        pltpu.make_async_copy(
                            tiles.at[slot].at[pl.ds(0, SUB)],
                            out_hbm.at[0].at[0].at[0],
                            sem_o[slot],
                        ).wait()

                pltpu.make_async_copy(
                    table_hbm.at[pl.ds(0, LANES)], rows.at[slot],
                    sem_g[slot],
                ).wait()

                @plsc.parallel_loop(0, D, 1, unroll=4)
                def tloop(c, slot=slot):
                    col = zeros + c
                    for sl0 in range(SUB):
                        vec = plsc.load_gather(
                            rows.at[slot], [rowidx[sl0], col]
                        )
                        tiles.at[slot].at[c][pl.ds(sl0 * L, L)] = vec

                for eh in range(EH):
                    pltpu.async_copy(
                        tiles.at[slot].at[pl.ds(eh * SUB, SUB)],
                        out_hbm.at[f].at[eh].at[sh],
                        sem_o[slot],
                    )

                @pl.when(r + 2 < nr)
                def _():
                    fire(r + 2, slot)
            return carry

        lax.fori_loop(0, nr // 2, body, 0)

        for slot in range(2):
            for _ in range(EH):
                pltpu.make_async_copy(
                    tiles.at[slot].at[pl.ds(0, SUB)],
                    out_hbm.at[0].at[0].at[0],
                    sem_o[slot],
                ).wait()

    return gather_kernel


def kernel(x, weight):
    batch, nf = x.shape
    V, D = weight.shape
    idx = x.reshape(batch * nf)
    out5d = _make_gather(batch, nf, V, D)(idx, weight)
    t = jnp.transpose(out5d, (2, 4, 0, 1, 3))
    return t.reshape(batch, nf, D)


# speedup vs baseline: 1.1512x; 1.0021x over previous
import functools

import jax
import jax.numpy as jnp
from jax import lax
from jax.experimental import pallas as pl
from jax.experimental.pallas import tpu as pltpu
from jax.experimental.pallas import tpu_sc as plsc

L = 16
SUB = 8
LANES = 128


@functools.lru_cache(maxsize=None)
def _make_gather(batch, nf, V, D):
    info = plsc.get_sparse_core_info()
    NC, NS = info.num_cores, info.num_subcores
    NW = NC * NS
    EH = D // SUB
    SH = batch // LANES
    assert SH % NW == 0
    spw = SH // NW
    r_per_w = (batch // NW) * nf
    blk = LANES * nf
    nr = spw * nf

    mesh = plsc.VectorSubcoreMesh(core_axis_name="c", subcore_axis_name="s")

    @functools.partial(
        pl.kernel,
        mesh=mesh,
        out_type=jax.ShapeDtypeStruct((nf, EH, SH, SUB, LANES), jnp.float32),
        scratch_types=[
            pltpu.VMEM((r_per_w,), jnp.int32),
            pltpu.VMEM((nr, LANES), jnp.int32),
            pltpu.VMEM((2, LANES, D), jnp.float32),
            pltpu.VMEM((2, EH * SUB, LANES), jnp.float32),
            pltpu.SemaphoreType.DMA,
            pltpu.SemaphoreType.DMA,
            pltpu.SemaphoreType.DMA,
            pltpu.SemaphoreType.DMA,
        ],
        compiler_params=pltpu.CompilerParams(
            use_tc_tiling_on_sc=False, needs_layout_passes=False
        ),
    )
    def gather_kernel(idx_hbm, table_hbm, out_hbm, idx_v, idx_fm, rows,
                      tiles, sem_g0, sem_g1, sem_o0, sem_o1):
        sem_g = [sem_g0, sem_g1]
        sem_o = [sem_o0, sem_o1]
        wid = lax.axis_index("s") * NC + lax.axis_index("c")
        pltpu.sync_copy(idx_hbm.at[pl.ds(wid * r_per_w, r_per_w)], idx_v)

        lane = lax.iota(jnp.int32, L)
        lane_nf = lane * nf
        zeros = lane * 0
        rowidx = [lane + sl0 * L for sl0 in range(SUB)]

        @plsc.parallel_loop(0, nr, 1, unroll=2)
        def rbody(r):
            shl = r // nf
            f = r % nf
            base = shl * blk + f
            for sl0 in range(SUB):
                vec = plsc.load_gather(
                    idx_v, [lane_nf + (base + sl0 * L * nf)]
                )
                idx_fm.at[r][pl.ds(sl0 * L, L)] = vec

        def fire(r, slot):
            pltpu.async_copy(
                table_hbm.at[idx_fm.at[r]], rows.at[slot], sem_g[slot]
            )

        fire(0, 0)
        fire(1, 1)

        def body(rr, carry):
            for slot in range(2):
                r = rr * 2 + slot
                shl = r // nf
                f = r % nf
                sh = wid * spw + shl

                @pl.when(rr >= 1)
                def _():
                    for _ in range(EH):
                        pltpu.make_async_copy(
                            tiles.at[slot].at[pl.ds(0, SUB)],
                            out_hbm.at[0].at[0].at[0],
                            sem_o[slot],
                        ).wait()

                pltpu.make_async_copy(
                    table_hbm.at[pl.ds(0, LANES)], rows.at[slot],
                    sem_g[slot],
                ).wait()

                @plsc.parallel_loop(0, D, 1, unroll=4)
                def tloop(c, slot=slot):
                    col = zeros + c
                    for sl0 in range(SUB):
                        vec = plsc.load_gather(
                            rows.at[slot], [rowidx[sl0], col]
                        )
                        tiles.at[slot].at[c][pl.ds(sl0 * L, L)] = vec

                for eh in range(EH):
                    pltpu.async_copy(
                        tiles.at[slot].at[pl.ds(eh * SUB, SUB)],
                        out_hbm.at[f].at[eh].at[sh],
                        sem_o[slot],
                    )

                @pl.when(r + 2 < nr)
                def _():
                    fire(r + 2, slot)
            return carry

        lax.fori_loop(0, nr // 2, body, 0)

        for slot in range(2):
            for _ in range(EH):
                pltpu.make_async_copy(
                    tiles.at[slot].at[pl.ds(0, SUB)],
                    out_hbm.at[0].at[0].at[0],
                    sem_o[slot],
                ).wait()

    return gather_kernel


def kernel(x, weight):
    batch, nf = x.shape
    V, D = weight.shape
    idx = x.reshape(batch * nf)
    out5d = _make_gather(batch, nf, V, D)(idx, weight)
    t = jnp.transpose(out5d, (2, 4, 0, 1, 3))
    return t.reshape(batch, nf, D)
